# baseline (speedup 1.0000x reference)
import jax
import jax.numpy as jnp
from jax.experimental import pallas as pl
from jax.experimental.pallas import tpu as pltpu

_B, _C, _H, _W = 8, 3, 384, 384
_PATCH = 16
_PD = _C * _PATCH * _PATCH
_D = 768
_N = (_H // _PATCH) * (_W // _PATCH)
_E = 3
_K = 5
_KP = 8
_ITERS = 10
_ROWS = _C * _H
_RG = _C * _PATCH
_BB = 4


def _dot(a, b, dims, prec):
    return jax.lax.dot_general(a, b, (dims, ((), ())),
                               preferred_element_type=jnp.float32,
                               precision=prec)


def _mm(a, b):
    return _dot(a, b, ((1,), (0,)), jax.lax.Precision.DEFAULT)


def _split2(a):
    hi = a.astype(jnp.bfloat16)
    return hi, (a - hi.astype(jnp.float32)).astype(jnp.bfloat16)


def _split3(a):
    hi = a.astype(jnp.bfloat16)
    r1 = a - hi.astype(jnp.float32)
    mid = r1.astype(jnp.bfloat16)
    lo = (r1 - mid.astype(jnp.float32)).astype(jnp.bfloat16)
    return hi, mid, lo


def _kmeans3(X0, X1, X2):
    NT, KT = 3 * _N, 3 * _KP
    X = jnp.concatenate([X0, X1, X2], axis=0)

    def init8(Xe):
        return jnp.concatenate(
            [Xe[24 * q:24 * q + 1] for q in range(_KP)], axis=0)

    centers = jnp.concatenate([init8(X0), init8(X1), init8(X2)])
    Xa, Xb, Xc = _split3(X)
    rowid = jax.lax.broadcasted_iota(jnp.int32, (KT, NT), 0)
    colblk = jnp.concatenate(
        [jnp.full((KT, _N), e, jnp.int32) for e in range(3)], axis=1)
    valid = (rowid >> 3 == colblk) & ((rowid & 7) < _K)
    ones_col = jnp.ones((NT, 1), jnp.bfloat16)
    xnorm = jnp.sum(X * X, axis=1, keepdims=True)
    xnT = xnorm.T
    for _ in range(_ITERS):
        cn = jnp.sum(centers * centers, axis=1, keepdims=True)
        xcT = _dot(centers, X, ((1,), (1,)),
                   jax.lax.Precision.DEFAULT)
        scoresT = (xnT - 2.0 * xcT) + cn
        scoresT = jnp.where(valid, scoresT, jnp.inf)
        mT = jnp.min(scoresT, axis=0, keepdims=True)
        maskedT = jnp.where(scoresT == mT, rowid, KT)
        aminT = jnp.min(maskedT, axis=0, keepdims=True)
        onehotT = (rowid == aminT).astype(jnp.bfloat16)
        sums = (_dot(onehotT, Xa, ((1,), (0,)), jax.lax.Precision.DEFAULT)
                + _dot(onehotT, Xb, ((1,), (0,)), jax.lax.Precision.DEFAULT)
                + _dot(onehotT, Xc, ((1,), (0,)), jax.lax.Precision.DEFAULT))
        counts = _dot(onehotT, ones_col, ((1,), (0,)),
                      jax.lax.Precision.DEFAULT)
        centers = sums / jnp.maximum(counts, 1.0)
    return centers


def _kfn(x_ref, w0_ref, w1_ref, w2_ref, r_ref, l_ref, m_ref, s_ref,
         feat_ref, cent_ref, psum_ref):
    b = pl.program_id(0)

    xb = x_ref[...].reshape(_BB * _ROWS, _W)
    xh, xl = _split2(xb)
    cs = (_mm(xh, r_ref[...]) + _mm(xl, r_ref[...]))
    ch, cl = _split2(cs)
    pb = (_mm(l_ref[...], ch) + _mm(l_ref[...], cl))
    psum_ref[pl.ds(b * _BB * _RG, _BB * _RG), :] = pb

    @pl.when(b == 0)
    def _():
        cols = []
        for c in range(_C):
            for i in range(_PATCH):
                a = x_ref[c, :, i, :]
                cols.append(jnp.concatenate(
                    [a[:, gw * _PATCH:(gw + 1) * _PATCH]
                     for gw in range(24)], axis=0))
        p0 = jnp.concatenate(cols, axis=1)
        Xs = [_mm(p0, w_ref[...])
              for w_ref in (w0_ref, w1_ref, w2_ref)]
        centers = _kmeans3(*Xs)
        for e in range(3):
            cent_ref[e] = centers[e * _KP:(e + 1) * _KP]

    @pl.when(b == _B // _BB - 1)
    def _():
        ps = psum_ref[...]
        t32 = jnp.concatenate([ps, ps], axis=1)
        t64 = jnp.concatenate([t32, t32], axis=1)
        t128 = jnp.concatenate([t64, t64], axis=1)
        t256 = jnp.concatenate([t128, t128], axis=1)
        t512 = jnp.concatenate([t256, t256], axis=1)
        t768 = jnp.concatenate([t512, t256], axis=1)
        P = t768 * m_ref[...]
        ph, plo = _split2(P)
        pbar = (_mm(s_ref[...], ph) + _mm(s_ref[...], plo)) * (1.0 / _N)
        for e, w_ref in enumerate((w0_ref, w1_ref, w2_ref)):
            feat_ref[e] = _mm(pbar, w_ref[...])


def kernel(x, W0, W1, W2):
    b, c, h, w = x.shape
    nh, nw = h // _PATCH, w // _PATCH
    xr = x.reshape(b * c, nh, _PATCH, w)

    wi = jax.lax.broadcasted_iota(jnp.int32, (w, _PATCH), 0)
    ji = jax.lax.broadcasted_iota(jnp.int32, (w, _PATCH), 1)
    Rm = (wi % _PATCH == ji).astype(jnp.bfloat16)
    ri = jax.lax.broadcasted_iota(jnp.int32, (_BB * _RG, _BB * _ROWS), 0)
    qi = jax.lax.broadcasted_iota(jnp.int32, (_BB * _RG, _BB * _ROWS), 1)
    Lm = ((qi // _ROWS == ri // _RG)
          & (qi % _ROWS // h == ri % _RG // _PATCH)
          & (qi % _PATCH == ri % _PATCH)).astype(jnp.bfloat16)
    gi = jax.lax.broadcasted_iota(jnp.int32, (_B * _RG, _PD), 0)
    pi = jax.lax.broadcasted_iota(jnp.int32, (_B * _RG, _PD), 1)
    Mm = (pi // _PATCH == gi % _RG).astype(jnp.float32)
    bi = jax.lax.broadcasted_iota(jnp.int32, (_B, _B * _RG), 0)
    si = jax.lax.broadcasted_iota(jnp.int32, (_B, _B * _RG), 1)
    Sel = (si // _RG == bi).astype(jnp.bfloat16)

    feat, cent = pl.pallas_call(
        _kfn,
        grid=(_B // _BB,),
        in_specs=[
            pl.BlockSpec((_C * _BB, nh, _PATCH, w), lambda i: (i, 0, 0, 0)),
            pl.BlockSpec((_PD, _D), lambda i: (0, 0)),
            pl.BlockSpec((_PD, _D), lambda i: (0, 0)),
            pl.BlockSpec((_PD, _D), lambda i: (0, 0)),
            pl.BlockSpec((w, _PATCH), lambda i: (0, 0)),
            pl.BlockSpec((_BB * _RG, _BB * _ROWS), lambda i: (0, 0)),
            pl.BlockSpec((_B * _RG, _PD), lambda i: (0, 0)),
            pl.BlockSpec((_B, _B * _RG), lambda i: (0, 0)),
        ],
        out_specs=[
            pl.BlockSpec((_E, _B, _D), lambda i: (0, 0, 0)),
            pl.BlockSpec((_E, _KP, _D), lambda i: (0, 0, 0)),
        ],
        out_shape=[
            jax.ShapeDtypeStruct((_E, _B, _D), jnp.float32),
            jax.ShapeDtypeStruct((_E, _KP, _D), jnp.float32),
        ],
        scratch_shapes=[pltpu.VMEM((_B * _RG, _PATCH), jnp.float32)],
        compiler_params=pltpu.CompilerParams(
            dimension_semantics=("arbitrary",)),
    )(xr, W0, W1, W2, Rm, Lm, Mm, Sel)

    features = feat
    features_local = cent[:, :_K][:, None]
    return (features, features_local)


# speedup vs baseline: 1.0537x; 1.0537x over previous
import jax
import jax.numpy as jnp
from jax.experimental import pallas as pl
from jax.experimental.pallas import tpu as pltpu

_B, _C, _H, _W = 8, 3, 384, 384
_PATCH = 16
_PD = _C * _PATCH * _PATCH
_D = 768
_N = (_H // _PATCH) * (_W // _PATCH)
_E = 3
_K = 5
_KP = 8
_ITERS = 10
_ROWS = _C * _H
_RG = _C * _PATCH
_BB = 2


def _dot(a, b, dims, prec):
    return jax.lax.dot_general(a, b, (dims, ((), ())),
                               preferred_element_type=jnp.float32,
                               precision=prec)


def _mm(a, b):
    return _dot(a, b, ((1,), (0,)), jax.lax.Precision.DEFAULT)


def _split2(a):
    hi = a.astype(jnp.bfloat16)
    return hi, (a - hi.astype(jnp.float32)).astype(jnp.bfloat16)


def _split3(a):
    hi = a.astype(jnp.bfloat16)
    r1 = a - hi.astype(jnp.float32)
    mid = r1.astype(jnp.bfloat16)
    lo = (r1 - mid.astype(jnp.float32)).astype(jnp.bfloat16)
    return hi, mid, lo


def _kmeans3(X0, X1, X2):
    NT, KT = 3 * _N, 3 * _KP
    X = jnp.concatenate([X0, X1, X2], axis=0)

    def init8(Xe):
        return jnp.concatenate(
            [Xe[24 * q:24 * q + 1] for q in range(_KP)], axis=0)

    centers = jnp.concatenate([init8(X0), init8(X1), init8(X2)])
    Xa, Xb, Xc = _split3(X)
    rowid = jax.lax.broadcasted_iota(jnp.int32, (KT, NT), 0)
    colblk = jnp.concatenate(
        [jnp.full((KT, _N), e, jnp.int32) for e in range(3)], axis=1)
    valid = (rowid >> 3 == colblk) & ((rowid & 7) < _K)
    ones_col = jnp.ones((NT, 1), jnp.bfloat16)
    xnorm = jnp.sum(X * X, axis=1, keepdims=True)
    xnT = xnorm.T
    for _ in range(_ITERS):
        cn = jnp.sum(centers * centers, axis=1, keepdims=True)
        xcT = _dot(centers, X, ((1,), (1,)),
                   jax.lax.Precision.DEFAULT)
        scoresT = (xnT - 2.0 * xcT) + cn
        scoresT = jnp.where(valid, scoresT, jnp.inf)
        mT = jnp.min(scoresT, axis=0, keepdims=True)
        maskedT = jnp.where(scoresT == mT, rowid, KT)
        aminT = jnp.min(maskedT, axis=0, keepdims=True)
        onehotT = (rowid == aminT).astype(jnp.bfloat16)
        sums = (_dot(onehotT, Xa, ((1,), (0,)), jax.lax.Precision.DEFAULT)
                + _dot(onehotT, Xb, ((1,), (0,)), jax.lax.Precision.DEFAULT)
                + _dot(onehotT, Xc, ((1,), (0,)), jax.lax.Precision.DEFAULT))
        counts = _dot(onehotT, ones_col, ((1,), (0,)),
                      jax.lax.Precision.DEFAULT)
        centers = sums / jnp.maximum(counts, 1.0)
    return centers


def _kfn(x_ref, w0_ref, w1_ref, w2_ref, r_ref, l_ref, m_ref, s_ref,
         feat_ref, cent_ref, psum_ref):
    b = pl.program_id(0)

    xb = x_ref[...].reshape(_BB * _ROWS, _W)
    xh, xl = _split2(xb)
    cs = (_mm(xh, r_ref[...]) + _mm(xl, r_ref[...]))
    ch, cl = _split2(cs)
    pb = (_mm(l_ref[...], ch) + _mm(l_ref[...], cl))
    psum_ref[pl.ds(b * _BB * _RG, _BB * _RG), :] = pb

    @pl.when(b == 0)
    def _():
        cols = []
        for c in range(_C):
            for i in range(_PATCH):
                a = x_ref[c, :, i, :]
                cols.append(jnp.concatenate(
                    [a[:, gw * _PATCH:(gw + 1) * _PATCH]
                     for gw in range(24)], axis=0))
        p0 = jnp.concatenate(cols, axis=1)
        Xs = [_mm(p0, w_ref[...])
              for w_ref in (w0_ref, w1_ref, w2_ref)]
        centers = _kmeans3(*Xs)
        for e in range(3):
            cent_ref[e] = centers[e * _KP:(e + 1) * _KP]

    @pl.when(b == _B // _BB - 1)
    def _():
        ps = psum_ref[...]
        t32 = jnp.concatenate([ps, ps], axis=1)
        t64 = jnp.concatenate([t32, t32], axis=1)
        t128 = jnp.concatenate([t64, t64], axis=1)
        t256 = jnp.concatenate([t128, t128], axis=1)
        t512 = jnp.concatenate([t256, t256], axis=1)
        t768 = jnp.concatenate([t512, t256], axis=1)
        P = t768 * m_ref[...]
        ph, plo = _split2(P)
        pbar = (_mm(s_ref[...], ph) + _mm(s_ref[...], plo)) * (1.0 / _N)
        for e, w_ref in enumerate((w0_ref, w1_ref, w2_ref)):
            feat_ref[e] = _mm(pbar, w_ref[...])


def kernel(x, W0, W1, W2):
    b, c, h, w = x.shape
    nh, nw = h // _PATCH, w // _PATCH
    xr = x.reshape(b * c, nh, _PATCH, w)

    wi = jax.lax.broadcasted_iota(jnp.int32, (w, _PATCH), 0)
    ji = jax.lax.broadcasted_iota(jnp.int32, (w, _PATCH), 1)
    Rm = (wi % _PATCH == ji).astype(jnp.bfloat16)
    ri = jax.lax.broadcasted_iota(jnp.int32, (_BB * _RG, _BB * _ROWS), 0)
    qi = jax.lax.broadcasted_iota(jnp.int32, (_BB * _RG, _BB * _ROWS), 1)
    Lm = ((qi // _ROWS == ri // _RG)
          & (qi % _ROWS // h == ri % _RG // _PATCH)
          & (qi % _PATCH == ri % _PATCH)).astype(jnp.bfloat16)
    gi = jax.lax.broadcasted_iota(jnp.int32, (_B * _RG, _PD), 0)
    pi = jax.lax.broadcasted_iota(jnp.int32, (_B * _RG, _PD), 1)
    Mm = (pi // _PATCH == gi % _RG).astype(jnp.float32)
    bi = jax.lax.broadcasted_iota(jnp.int32, (_B, _B * _RG), 0)
    si = jax.lax.broadcasted_iota(jnp.int32, (_B, _B * _RG), 1)
    Sel = (si // _RG == bi).astype(jnp.bfloat16)

    feat, cent = pl.pallas_call(
        _kfn,
        grid=(_B // _BB,),
        in_specs=[
            pl.BlockSpec((_C * _BB, nh, _PATCH, w), lambda i: (i, 0, 0, 0)),
            pl.BlockSpec((_PD, _D), lambda i: (0, 0)),
            pl.BlockSpec((_PD, _D), lambda i: (0, 0)),
            pl.BlockSpec((_PD, _D), lambda i: (0, 0)),
            pl.BlockSpec((w, _PATCH), lambda i: (0, 0)),
            pl.BlockSpec((_BB * _RG, _BB * _ROWS), lambda i: (0, 0)),
            pl.BlockSpec((_B * _RG, _PD), lambda i: (0, 0)),
            pl.BlockSpec((_B, _B * _RG), lambda i: (0, 0)),
        ],
        out_specs=[
            pl.BlockSpec((_E, _B, _D), lambda i: (0, 0, 0)),
            pl.BlockSpec((_E, _KP, _D), lambda i: (0, 0, 0)),
        ],
        out_shape=[
            jax.ShapeDtypeStruct((_E, _B, _D), jnp.float32),
            jax.ShapeDtypeStruct((_E, _KP, _D), jnp.float32),
        ],
        scratch_shapes=[pltpu.VMEM((_B * _RG, _PATCH), jnp.float32)],
        compiler_params=pltpu.CompilerParams(
            dimension_semantics=("arbitrary",)),
    )(xr, W0, W1, W2, Rm, Lm, Mm, Sel)

    features = feat
    features_local = cent[:, :_K][:, None]
    return (features, features_local)
